# baseline (speedup 1.0000x reference)
import functools

import jax
import jax.numpy as jnp
from jax import lax
from jax.experimental import pallas as pl
from jax.experimental.pallas import tpu as pltpu
from jax.experimental.pallas import tpu_sc as plsc

B = 16384
D = 64
NC = 2
NS = 16
NW = NC * NS
BPW = B // NW
CH = 128
NCH = BPW // CH

@functools.cache
def _make_sc_gather():
    mesh = plsc.VectorSubcoreMesh(core_axis_name="c", subcore_axis_name="s")

    @functools.partial(
        pl.kernel,
        out_type=[jax.ShapeDtypeStruct((B, D), jnp.float32) for _ in range(4)],
        mesh=mesh,
        scratch_types=[
            pltpu.VMEM((NCH, CH), jnp.int32),
            pltpu.VMEM((NCH, CH), jnp.int32),
            pltpu.VMEM((BPW, D), jnp.float32),
            pltpu.SemaphoreType.DMA,
        ],
        compiler_params=pltpu.CompilerParams(use_tc_tiling_on_sc=False),
    )
    def _sc_gather(umlp, imlp, umf, imf, uidx, iidx,
                   out_umlp, out_imlp, out_umf, out_imf,
                   uidx_v, iidx_v, buf, sem):
        wid = lax.axis_index("s") * NC + lax.axis_index("c")
        base = wid * BPW
        pltpu.sync_copy(uidx.at[wid], uidx_v)
        pltpu.sync_copy(iidx.at[wid], iidx_v)
        for table, idx_v, out in (
            (umlp, uidx_v, out_umlp),
            (imlp, iidx_v, out_imlp),
            (umf, uidx_v, out_umf),
            (imf, iidx_v, out_imf),
        ):
            descs = []
            for c in range(NCH):
                descs.append(
                    pltpu.async_copy(table.at[idx_v.at[c]],
                                     buf.at[pl.ds(c * CH, CH)], sem))
            for d in descs:
                d.wait()
            pltpu.sync_copy(buf, out.at[pl.ds(base, BPW)])

    return _sc_gather


def _tc_body(u_ref, i_ref, uf_ref, if_ref,
             w0u_ref, w0i_ref, b0_ref, w1_ref, b1_ref, w2_ref, b2_ref,
             wamlp_ref, wamf_ref, ba_ref, out_ref):
    h = jnp.dot(u_ref[...], w0u_ref[...], preferred_element_type=jnp.float32)
    h += jnp.dot(i_ref[...], w0i_ref[...], preferred_element_type=jnp.float32)
    h = jnp.maximum(h + b0_ref[...], 0.0)
    h = jnp.maximum(
        jnp.dot(h, w1_ref[...], preferred_element_type=jnp.float32) + b1_ref[...], 0.0)
    h = jnp.maximum(
        jnp.dot(h, w2_ref[...], preferred_element_type=jnp.float32) + b2_ref[...], 0.0)
    mf = uf_ref[...] * if_ref[...]
    logit = (jnp.dot(h, wamlp_ref[...], preferred_element_type=jnp.float32)
             + jnp.dot(mf, wamf_ref[...], preferred_element_type=jnp.float32)
             + ba_ref[...])
    out_ref[...] = jax.nn.sigmoid(logit)


def kernel(user_indices, item_indices, user_mlp, item_mlp, user_mf, item_mf,
           W0, b0, W1, b1, W2, b2, Wa, ba):
    uidx = user_indices.astype(jnp.int32).reshape(NW, NCH, CH)
    iidx = item_indices.astype(jnp.int32).reshape(NW, NCH, CH)
    u_rows, i_rows, uf_rows, if_rows = _make_sc_gather()(
        user_mlp, item_mlp, user_mf, item_mf, uidx, iidx)

    w0u = W0.T[:D]
    w0i = W0.T[D:]
    w1 = W1.T
    w2 = W2.T
    wamlp = Wa[:, :32].T
    wamf = Wa[:, 32:].T
    b0r = b0.reshape(1, -1)
    b1r = b1.reshape(1, -1)
    b2r = b2.reshape(1, -1)
    bar = ba.reshape(1, 1)

    BT = 1024
    nblk = B // BT
    row_spec = pl.BlockSpec((BT, D), lambda b: (b, 0))
    full = lambda shape: pl.BlockSpec(shape, lambda b: tuple(0 for _ in shape))
    out = pl.pallas_call(
        _tc_body,
        grid=(nblk,),
        in_specs=[
            row_spec, row_spec, row_spec, row_spec,
            full((D, 128)), full((D, 128)), full((1, 128)),
            full((128, D)), full((1, D)),
            full((D, 32)), full((1, 32)),
            full((32, 1)), full((D, 1)), full((1, 1)),
        ],
        out_specs=pl.BlockSpec((BT, 1), lambda b: (b, 0)),
        out_shape=jax.ShapeDtypeStruct((B, 1), jnp.float32),
    )(u_rows, i_rows, uf_rows, if_rows,
      w0u, w0i, b0r, w1, b1r, w2, b2r, wamlp, wamf, bar)
    return out


# speedup vs baseline: 1.3984x; 1.3984x over previous
import functools

import jax
import jax.numpy as jnp
from jax import lax
from jax.experimental import pallas as pl
from jax.experimental.pallas import tpu as pltpu
from jax.experimental.pallas import tpu_sc as plsc

B = 16384
D = 64
NC = 2
NS = 16
NW = NC * NS
BPW = B // NW
CH = 128
NCH = BPW // CH

@functools.cache
def _make_sc_gather():
    mesh = plsc.VectorSubcoreMesh(core_axis_name="c", subcore_axis_name="s")

    @functools.partial(
        pl.kernel,
        out_type=[jax.ShapeDtypeStruct((B, D), jnp.float32) for _ in range(4)],
        mesh=mesh,
        scratch_types=[
            pltpu.VMEM((BPW,), jnp.int32),
            pltpu.VMEM((BPW,), jnp.int32),
            pltpu.VMEM((BPW, D), jnp.float32),
            pltpu.SemaphoreType.DMA,
        ],
    )
    def _sc_gather(umlp, imlp, umf, imf, uidx, iidx,
                   out_umlp, out_imlp, out_umf, out_imf,
                   uidx_v, iidx_v, buf, sem):
        wid = lax.axis_index("s") * NC + lax.axis_index("c")
        base = wid * BPW
        pltpu.sync_copy(uidx.at[wid], uidx_v)
        pltpu.sync_copy(iidx.at[wid], iidx_v)
        for table, idx_v, out in (
            (umlp, uidx_v, out_umlp),
            (imlp, iidx_v, out_imlp),
            (umf, uidx_v, out_umf),
            (imf, iidx_v, out_imf),
        ):
            def group_dma(g, _):
                vec = idx_v[pl.ds(g * 16, 16)]
                for k in range(16):
                    pltpu.async_copy(table.at[vec[k]], buf.at[g * 16 + k], sem)
                return 0

            lax.fori_loop(0, BPW // 16, group_dma, 0)
            pltpu.make_async_copy(table.at[pl.ds(0, BPW)], buf, sem).wait()
            pltpu.sync_copy(buf, out.at[pl.ds(base, BPW)])

    return _sc_gather


def _tc_body(u_ref, i_ref, uf_ref, if_ref,
             w0u_ref, w0i_ref, b0_ref, w1_ref, b1_ref, w2_ref, b2_ref,
             wamlp_ref, wamf_ref, ba_ref, out_ref):
    h = jnp.dot(u_ref[...], w0u_ref[...], preferred_element_type=jnp.float32)
    h += jnp.dot(i_ref[...], w0i_ref[...], preferred_element_type=jnp.float32)
    h = jnp.maximum(h + b0_ref[...], 0.0)
    h = jnp.maximum(
        jnp.dot(h, w1_ref[...], preferred_element_type=jnp.float32) + b1_ref[...], 0.0)
    h = jnp.maximum(
        jnp.dot(h, w2_ref[...], preferred_element_type=jnp.float32) + b2_ref[...], 0.0)
    mf = uf_ref[...] * if_ref[...]
    logit = (jnp.dot(h, wamlp_ref[...], preferred_element_type=jnp.float32)
             + jnp.dot(mf, wamf_ref[...], preferred_element_type=jnp.float32)
             + ba_ref[...])
    out_ref[...] = jax.nn.sigmoid(logit)


def kernel(user_indices, item_indices, user_mlp, item_mlp, user_mf, item_mf,
           W0, b0, W1, b1, W2, b2, Wa, ba):
    uidx = user_indices.astype(jnp.int32).reshape(NW, BPW)
    iidx = item_indices.astype(jnp.int32).reshape(NW, BPW)
    u_rows, i_rows, uf_rows, if_rows = _make_sc_gather()(
        user_mlp, item_mlp, user_mf, item_mf, uidx, iidx)

    w0u = W0.T[:D]
    w0i = W0.T[D:]
    w1 = W1.T
    w2 = W2.T
    wamlp = Wa[:, :32].T
    wamf = Wa[:, 32:].T
    b0r = b0.reshape(1, -1)
    b1r = b1.reshape(1, -1)
    b2r = b2.reshape(1, -1)
    bar = ba.reshape(1, 1)

    BT = 1024
    nblk = B // BT
    row_spec = pl.BlockSpec((BT, D), lambda b: (b, 0))
    full = lambda shape: pl.BlockSpec(shape, lambda b: tuple(0 for _ in shape))
    out = pl.pallas_call(
        _tc_body,
        grid=(nblk,),
        in_specs=[
            row_spec, row_spec, row_spec, row_spec,
            full((D, 128)), full((D, 128)), full((1, 128)),
            full((128, D)), full((1, D)),
            full((D, 32)), full((1, 32)),
            full((32, 1)), full((D, 1)), full((1, 1)),
        ],
        out_specs=pl.BlockSpec((BT, 1), lambda b: (b, 0)),
        out_shape=jax.ShapeDtypeStruct((B, 1), jnp.float32),
    )(u_rows, i_rows, uf_rows, if_rows,
      w0u, w0i, b0r, w1, b1r, w2, b2r, wamlp, wamf, bar)
    return out
